# baseline (speedup 1.0000x reference)
import functools

import jax
import jax.numpy as jnp
from jax import lax
from jax.experimental import pallas as pl
from jax.experimental.pallas import tpu as pltpu
from jax.experimental.pallas import tpu_sc as plsc

E = 8
TOPK = 2
TILE = 256
D = 1024
F = 4096
MOE_D = 2048


def _gmm_body(meta_ref, xd_ref, w13_hbm, w2_hbm, wrow_ref, y_ref,
              w13_buf, w2_buf, sem13, sem2):
    i = pl.program_id(0)
    e = meta_ref[0, i]
    p = meta_ref[1, i]
    first = meta_ref[2, i]
    nxt = meta_ref[3, i]
    hasn = meta_ref[4, i]

    @pl.when(i == 0)
    def _():
        pltpu.make_async_copy(w13_hbm.at[e], w13_buf.at[p], sem13.at[p]).start()
        pltpu.make_async_copy(w2_hbm.at[e], w2_buf.at[p], sem2.at[p]).start()

    @pl.when((first == 1) & (hasn == 1))
    def _():
        q = 1 - p
        pltpu.make_async_copy(w13_hbm.at[nxt], w13_buf.at[q], sem13.at[q]).start()
        pltpu.make_async_copy(w2_hbm.at[nxt], w2_buf.at[q], sem2.at[q]).start()

    @pl.when(first == 1)
    def _():
        pltpu.make_async_copy(w13_hbm.at[e], w13_buf.at[p], sem13.at[p]).wait()
        pltpu.make_async_copy(w2_hbm.at[e], w2_buf.at[p], sem2.at[p]).wait()

    h = jnp.dot(xd_ref[...], w13_buf[p], preferred_element_type=jnp.float32)
    gate = h[:, :MOE_D]
    up = h[:, MOE_D:]
    a = jax.nn.silu(gate) * up
    y = jnp.dot(a, w2_buf[p], preferred_element_type=jnp.float32)
    y_ref[...] = y * wrow_ref[...]


def _seg_meta(eot, nt):
    idx = jnp.arange(nt, dtype=jnp.int32)
    prev = jnp.concatenate([eot[:1] - 1, eot[:-1]])
    first = (eot != prev).astype(jnp.int32)
    seg = jnp.cumsum(first) - 1
    par = (seg % 2).astype(jnp.int32)
    diff = (eot[None, :] != eot[:, None]) & (idx[None, :] > idx[:, None])
    hasn = jnp.any(diff, axis=1)
    j = jnp.argmax(diff, axis=1)
    nxt = jnp.where(hasn, eot[j], eot)
    return jnp.stack([eot, par, first, nxt, hasn.astype(jnp.int32)])


def _grouped_mlp(eot, xd, w13, w2, wrow, nt):
    meta = _seg_meta(eot, nt)
    grid_spec = pltpu.PrefetchScalarGridSpec(
        num_scalar_prefetch=1,
        grid=(nt,),
        in_specs=[
            pl.BlockSpec((TILE, D), lambda i, meta: (i, 0)),
            pl.BlockSpec(memory_space=pl.ANY),
            pl.BlockSpec(memory_space=pl.ANY),
            pl.BlockSpec((TILE, 1), lambda i, meta: (i, 0)),
        ],
        out_specs=pl.BlockSpec((TILE, D), lambda i, meta: (i, 0)),
        scratch_shapes=[
            pltpu.VMEM((2, D, F), jnp.float32),
            pltpu.VMEM((2, MOE_D, D), jnp.float32),
            pltpu.SemaphoreType.DMA((2,)),
            pltpu.SemaphoreType.DMA((2,)),
        ],
    )
    return pl.pallas_call(
        _gmm_body,
        grid_spec=grid_spec,
        out_shape=jax.ShapeDtypeStruct((nt * TILE, D), jnp.float32),
        compiler_params=pltpu.CompilerParams(
            vmem_limit_bytes=110 * 1024 * 1024,
        ),
    )(meta, xd, w13, w2, wrow)


def _routing_body(x_ref, rp_ref, e1_ref, e2_ref, w1_ref, w2_ref):
    l = jnp.dot(x_ref[...], rp_ref[...], preferred_element_type=jnp.float32)
    col = jax.lax.broadcasted_iota(jnp.int32, l.shape, 1)
    lm = jnp.where(col < E, l, -jnp.inf)
    m1 = jnp.max(lm, axis=1, keepdims=True)
    i1 = jnp.min(jnp.where(lm == m1, col, 128), axis=1, keepdims=True)
    l2 = jnp.where(col == i1, -jnp.inf, lm)
    m2 = jnp.max(l2, axis=1, keepdims=True)
    i2 = jnp.min(jnp.where(l2 == m2, col, 128), axis=1, keepdims=True)
    s = jnp.exp(m2 - m1)
    w1 = 1.0 / (1.0 + s)
    e1_ref[...] = i1
    e2_ref[...] = i2
    w1_ref[...] = w1
    w2_ref[...] = 1.0 - w1


def _routing(x_flat, router, tokens):
    rp = jnp.zeros((D, 128), jnp.float32).at[:, :E].set(router)
    return pl.pallas_call(
        _routing_body,
        out_shape=[
            jax.ShapeDtypeStruct((tokens, 1), jnp.int32),
            jax.ShapeDtypeStruct((tokens, 1), jnp.int32),
            jax.ShapeDtypeStruct((tokens, 1), jnp.float32),
            jax.ShapeDtypeStruct((tokens, 1), jnp.float32),
        ],
    )(x_flat, rp)


_NC = 2
_NS = 16
_NW = _NC * _NS


def _sc_gather(x_flat, src, r_pad):
    per_w = r_pad // _NW
    chunks = [(o, min(32, per_w - o)) for o in range(0, per_w, 32)]
    mesh = plsc.VectorSubcoreMesh(core_axis_name="c", subcore_axis_name="s")

    @functools.partial(
        pl.kernel, mesh=mesh,
        out_type=jax.ShapeDtypeStruct((r_pad, D), jnp.float32),
        scratch_types=[
            pltpu.VMEM((per_w,), jnp.int32),
            pltpu.VMEM((2, 32, D), jnp.float32),
            pltpu.SemaphoreType.DMA((2,)),
            pltpu.SemaphoreType.DMA((2,)),
        ],
    )
    def k(x_hbm, src_hbm, xd_hbm, idx_v, rows_v, gsem, osem):
        wid = lax.axis_index("s") * _NC + lax.axis_index("c")
        base = wid * per_w
        pltpu.sync_copy(src_hbm.at[pl.ds(base, per_w)], idx_v)
        n = len(chunks)
        for ci, (off, sz) in enumerate(chunks):
            p = ci % 2
            if ci >= 2:
                po, psz = chunks[ci - 2]
                pltpu.make_async_copy(
                    rows_v.at[p, pl.ds(0, psz)],
                    xd_hbm.at[pl.ds(base + po, psz)], osem.at[p]).wait()
            g = pltpu.make_async_copy(
                x_hbm.at[idx_v.at[pl.ds(off, sz)]],
                rows_v.at[p, pl.ds(0, sz)], gsem.at[p])
            g.start()
            g.wait()
            pltpu.make_async_copy(
                rows_v.at[p, pl.ds(0, sz)],
                xd_hbm.at[pl.ds(base + off, sz)], osem.at[p]).start()
        for ci in range(max(n - 2, 0), n):
            p = ci % 2
            off, sz = chunks[ci]
            pltpu.make_async_copy(
                rows_v.at[p, pl.ds(0, sz)],
                xd_hbm.at[pl.ds(base + off, sz)], osem.at[p]).wait()

    return k(x_flat, src)


def _sc_collect(y, pos, tokens):
    per_w = tokens // _NW
    mesh = plsc.VectorSubcoreMesh(core_axis_name="c", subcore_axis_name="s")

    @functools.partial(
        pl.kernel, mesh=mesh,
        out_type=jax.ShapeDtypeStruct((tokens, D), jnp.float32),
        scratch_types=[
            pltpu.VMEM((32,), jnp.int32),
            pltpu.VMEM((32,), jnp.int32),
            pltpu.VMEM((32, D), jnp.float32),
            pltpu.VMEM((32, D), jnp.float32),
            pltpu.VMEM((32, D), jnp.float32),
            pltpu.SemaphoreType.DMA,
        ],
    )
    def k(y_hbm, pos_hbm, out_hbm, p1_v, p2_v, r1_v, r2_v, o_v, sem):
        wid = lax.axis_index("s") * _NC + lax.axis_index("c")
        base = wid * per_w
        for c in range(per_w // 32):
            tok0 = base + c * 32
            pltpu.sync_copy(pos_hbm.at[pl.ds(tok0, 32)], p1_v)
            pltpu.sync_copy(pos_hbm.at[pl.ds(tokens + tok0, 32)], p2_v)
            g1 = pltpu.make_async_copy(y_hbm.at[p1_v], r1_v, sem)
            g2 = pltpu.make_async_copy(y_hbm.at[p2_v], r2_v, sem)
            g1.start()
            g2.start()
            g1.wait()
            g2.wait()

            def rbody(r, carry):
                for l in range(D // 16):
                    sl = pl.ds(l * 16, 16)
                    o_v[r, sl] = r1_v[r, sl] + r2_v[r, sl]
                return carry

            lax.fori_loop(0, 32, rbody, 0)
            pltpu.sync_copy(o_v, out_hbm.at[pl.ds(tok0, 32)])

    return k(y, pos)


def kernel(x, moe_router, moe_w13, moe_w2):
    b, s, d = x.shape
    tokens = b * s
    x_flat = x.reshape(tokens, d)
    nt = tokens * TOPK // TILE + E - 1
    r_pad = nt * TILE

    e1, e2, w1, w2 = _routing(x_flat, moe_router, tokens)
    e_slot = jnp.concatenate([e1[:, 0], e2[:, 0]])
    w_slot = jnp.concatenate([w1[:, 0], w2[:, 0]])
    oh = (e_slot[:, None] == jnp.arange(E)[None, :]).astype(jnp.int32)
    pref = jnp.cumsum(oh, axis=0)
    cnt = pref[-1]
    padded = ((cnt + TILE - 1) // TILE) * TILE
    pad_off = jnp.concatenate([jnp.zeros((1,), jnp.int32),
                               jnp.cumsum(padded)[:-1].astype(jnp.int32)])
    rank = jnp.sum(pref * oh, axis=1) - 1
    pos = pad_off[e_slot] + rank
    src = jnp.zeros((r_pad,), jnp.int32).at[pos].set(
        jnp.arange(tokens * TOPK, dtype=jnp.int32) % tokens)
    wdisp = jnp.zeros((r_pad,), jnp.float32).at[pos].set(w_slot)
    pad_end = (pad_off + padded).astype(jnp.int32)
    tile_start = jnp.arange(nt, dtype=jnp.int32) * TILE
    eot = jnp.minimum(
        jnp.sum((pad_end[None, :] <= tile_start[:, None]).astype(jnp.int32),
                axis=1), E - 1).astype(jnp.int32)

    xd = _sc_gather(x_flat, src, r_pad)

    y = _grouped_mlp(eot, xd, moe_w13, moe_w2, wdisp[:, None], nt)

    out_flat = _sc_collect(y, pos.astype(jnp.int32), tokens)
    return out_flat.reshape(b, s, d)


# speedup vs baseline: 1.2412x; 1.2412x over previous
import functools

import jax
import jax.numpy as jnp
from jax import lax
from jax.experimental import pallas as pl
from jax.experimental.pallas import tpu as pltpu
from jax.experimental.pallas import tpu_sc as plsc

E = 8
TOPK = 2
TILE = 256
D = 1024
F = 4096
MOE_D = 2048


def _gmm_body(meta_ref, xd_ref, w13_hbm, w2_hbm, wrow_ref, y_ref,
              w13_buf, w2_buf, sem13, sem2):
    i = pl.program_id(0)
    e = meta_ref[0, i]
    p = meta_ref[1, i]
    first = meta_ref[2, i]
    nxt = meta_ref[3, i]
    hasn = meta_ref[4, i]

    @pl.when(i == 0)
    def _():
        pltpu.make_async_copy(w13_hbm.at[e], w13_buf.at[p], sem13.at[p]).start()
        pltpu.make_async_copy(w2_hbm.at[e], w2_buf.at[p], sem2.at[p]).start()

    @pl.when((first == 1) & (hasn == 1))
    def _():
        q = 1 - p
        pltpu.make_async_copy(w13_hbm.at[nxt], w13_buf.at[q], sem13.at[q]).start()
        pltpu.make_async_copy(w2_hbm.at[nxt], w2_buf.at[q], sem2.at[q]).start()

    @pl.when(first == 1)
    def _():
        pltpu.make_async_copy(w13_hbm.at[e], w13_buf.at[p], sem13.at[p]).wait()
        pltpu.make_async_copy(w2_hbm.at[e], w2_buf.at[p], sem2.at[p]).wait()

    h = jnp.dot(xd_ref[...], w13_buf[p], preferred_element_type=jnp.float32)
    gate = h[:, :MOE_D]
    up = h[:, MOE_D:]
    a = jax.nn.silu(gate) * up
    y = jnp.dot(a, w2_buf[p], preferred_element_type=jnp.float32)
    y_ref[...] = y * wrow_ref[...]


def _seg_meta(eot, nt):
    idx = jnp.arange(nt, dtype=jnp.int32)
    prev = jnp.concatenate([eot[:1] - 1, eot[:-1]])
    first = (eot != prev).astype(jnp.int32)
    seg = jnp.cumsum(first) - 1
    par = (seg % 2).astype(jnp.int32)
    diff = (eot[None, :] != eot[:, None]) & (idx[None, :] > idx[:, None])
    hasn = jnp.any(diff, axis=1)
    j = jnp.argmax(diff, axis=1)
    nxt = jnp.where(hasn, eot[j], eot)
    return jnp.stack([eot, par, first, nxt, hasn.astype(jnp.int32)])


def _grouped_mlp(eot, xd, w13, w2, wrow, nt):
    meta = _seg_meta(eot, nt)
    grid_spec = pltpu.PrefetchScalarGridSpec(
        num_scalar_prefetch=1,
        grid=(nt,),
        in_specs=[
            pl.BlockSpec((TILE, D), lambda i, meta: (i, 0)),
            pl.BlockSpec(memory_space=pl.ANY),
            pl.BlockSpec(memory_space=pl.ANY),
            pl.BlockSpec((TILE, 1), lambda i, meta: (i, 0)),
        ],
        out_specs=pl.BlockSpec((TILE, D), lambda i, meta: (i, 0)),
        scratch_shapes=[
            pltpu.VMEM((2, D, F), jnp.float32),
            pltpu.VMEM((2, MOE_D, D), jnp.float32),
            pltpu.SemaphoreType.DMA((2,)),
            pltpu.SemaphoreType.DMA((2,)),
        ],
    )
    return pl.pallas_call(
        _gmm_body,
        grid_spec=grid_spec,
        out_shape=jax.ShapeDtypeStruct((nt * TILE, D), jnp.float32),
        compiler_params=pltpu.CompilerParams(
            vmem_limit_bytes=110 * 1024 * 1024,
        ),
    )(meta, xd, w13, w2, wrow)


def _routing_body(x_ref, rp_ref, e1_ref, e2_ref, w1_ref, w2_ref):
    l = jnp.dot(x_ref[...], rp_ref[...], preferred_element_type=jnp.float32)
    col = jax.lax.broadcasted_iota(jnp.int32, l.shape, 1)
    lm = jnp.where(col < E, l, -jnp.inf)
    m1 = jnp.max(lm, axis=1, keepdims=True)
    i1 = jnp.min(jnp.where(lm == m1, col, 128), axis=1, keepdims=True)
    l2 = jnp.where(col == i1, -jnp.inf, lm)
    m2 = jnp.max(l2, axis=1, keepdims=True)
    i2 = jnp.min(jnp.where(l2 == m2, col, 128), axis=1, keepdims=True)
    s = jnp.exp(m2 - m1)
    w1 = 1.0 / (1.0 + s)
    e1_ref[...] = i1
    e2_ref[...] = i2
    w1_ref[...] = w1
    w2_ref[...] = 1.0 - w1


def _routing(x_flat, router, tokens):
    rp = jnp.zeros((D, 128), jnp.float32).at[:, :E].set(router)
    return pl.pallas_call(
        _routing_body,
        out_shape=[
            jax.ShapeDtypeStruct((tokens, 1), jnp.int32),
            jax.ShapeDtypeStruct((tokens, 1), jnp.int32),
            jax.ShapeDtypeStruct((tokens, 1), jnp.float32),
            jax.ShapeDtypeStruct((tokens, 1), jnp.float32),
        ],
    )(x_flat, rp)


_NC = 2
_NS = 16
_NW = _NC * _NS


def _sc_gather(x_flat, src, r_pad):
    per_w = r_pad // _NW
    chunks = [(o, min(32, per_w - o)) for o in range(0, per_w, 32)]
    mesh = plsc.VectorSubcoreMesh(core_axis_name="c", subcore_axis_name="s")

    @functools.partial(
        pl.kernel, mesh=mesh,
        out_type=jax.ShapeDtypeStruct((r_pad, D), jnp.float32),
        scratch_types=[
            pltpu.VMEM((per_w,), jnp.int32),
            pltpu.VMEM((2, 32, D), jnp.float32),
            pltpu.SemaphoreType.DMA((2,)),
            pltpu.SemaphoreType.DMA((2,)),
        ],
    )
    def k(x_hbm, src_hbm, xd_hbm, idx_v, rows_v, gsem, osem):
        wid = lax.axis_index("s") * _NC + lax.axis_index("c")
        base = wid * per_w
        pltpu.sync_copy(src_hbm.at[pl.ds(base, per_w)], idx_v)
        n = len(chunks)
        for ci, (off, sz) in enumerate(chunks):
            p = ci % 2
            if ci >= 2:
                po, psz = chunks[ci - 2]
                pltpu.make_async_copy(
                    rows_v.at[p, pl.ds(0, psz)],
                    xd_hbm.at[pl.ds(base + po, psz)], osem.at[p]).wait()
            g = pltpu.make_async_copy(
                x_hbm.at[idx_v.at[pl.ds(off, sz)]],
                rows_v.at[p, pl.ds(0, sz)], gsem.at[p])
            g.start()
            g.wait()
            pltpu.make_async_copy(
                rows_v.at[p, pl.ds(0, sz)],
                xd_hbm.at[pl.ds(base + off, sz)], osem.at[p]).start()
        for ci in range(max(n - 2, 0), n):
            p = ci % 2
            off, sz = chunks[ci]
            pltpu.make_async_copy(
                rows_v.at[p, pl.ds(0, sz)],
                xd_hbm.at[pl.ds(base + off, sz)], osem.at[p]).wait()

    return k(x_flat, src)


def _sc_collect(y, pos, tokens):
    per_w = tokens // _NW
    mesh = plsc.VectorSubcoreMesh(core_axis_name="c", subcore_axis_name="s")

    @functools.partial(
        pl.kernel, mesh=mesh,
        out_type=jax.ShapeDtypeStruct((tokens, D), jnp.float32),
        scratch_types=[
            pltpu.VMEM((32,), jnp.int32),
            pltpu.VMEM((32,), jnp.int32),
            pltpu.VMEM((32, D), jnp.float32),
            pltpu.VMEM((32, D), jnp.float32),
            pltpu.VMEM((32, D), jnp.float32),
            pltpu.SemaphoreType.DMA,
        ],
    )
    def k(y_hbm, pos_hbm, out_hbm, p1_v, p2_v, r1_v, r2_v, o_v, sem):
        wid = lax.axis_index("s") * _NC + lax.axis_index("c")
        base = wid * per_w
        for c in range(per_w // 32):
            tok0 = base + c * 32
            pltpu.sync_copy(pos_hbm.at[pl.ds(tok0, 32)], p1_v)
            pltpu.sync_copy(pos_hbm.at[pl.ds(tokens + tok0, 32)], p2_v)
            g1 = pltpu.make_async_copy(y_hbm.at[p1_v], r1_v, sem)
            g2 = pltpu.make_async_copy(y_hbm.at[p2_v], r2_v, sem)
            g1.start()
            g2.start()
            g1.wait()
            g2.wait()

            def rbody(r, carry):
                for l in range(D // 16):
                    sl = pl.ds(l * 16, 16)
                    o_v[r, sl] = r1_v[r, sl] + r2_v[r, sl]
                return carry

            lax.fori_loop(0, 32, rbody, 0)
            pltpu.sync_copy(o_v, out_hbm.at[pl.ds(tok0, 32)])

    return k(y, pos)


def kernel(x, moe_router, moe_w13, moe_w2):
    b, s, d = x.shape
    tokens = b * s
    x_flat = x.reshape(tokens, d)
    nt = tokens * TOPK // TILE + E - 1
    r_pad = nt * TILE

    e1, e2, w1, w2 = _routing(x_flat, moe_router, tokens)
    e_slot = jnp.concatenate([e1[:, 0], e2[:, 0]])
    w_slot = jnp.concatenate([w1[:, 0], w2[:, 0]])
    oh = (e_slot[:, None] == jnp.arange(E)[None, :]).astype(jnp.int32)
    pref = jnp.cumsum(oh, axis=0)
    cnt = pref[-1]
    padded = ((cnt + TILE - 1) // TILE) * TILE
    pad_off = jnp.concatenate([jnp.zeros((1,), jnp.int32),
                               jnp.cumsum(padded)[:-1].astype(jnp.int32)])
    rank = jnp.sum(pref * oh, axis=1) - 1
    pos = pad_off[e_slot] + rank
    src = jnp.zeros((r_pad,), jnp.int32).at[pos].set(
        jnp.arange(tokens * TOPK, dtype=jnp.int32) % tokens)
    wdisp = jnp.zeros((r_pad,), jnp.float32).at[pos].set(w_slot)
    pad_end = (pad_off + padded).astype(jnp.int32)
    tile_start = jnp.arange(nt, dtype=jnp.int32) * TILE
    eot = jnp.minimum(
        jnp.sum((pad_end[None, :] <= tile_start[:, None]).astype(jnp.int32),
                axis=1), E - 1).astype(jnp.int32)

    xd = x_flat[src]

    y = _grouped_mlp(eot, xd, moe_w13, moe_w2, wdisp[:, None], nt)

    out_flat = _sc_collect(y, pos.astype(jnp.int32), tokens)
    return out_flat.reshape(b, s, d)


# speedup vs baseline: 1.2434x; 1.0018x over previous
import functools

import jax
import jax.numpy as jnp
from jax import lax
from jax.experimental import pallas as pl
from jax.experimental.pallas import tpu as pltpu
from jax.experimental.pallas import tpu_sc as plsc

E = 8
TOPK = 2
TILE = 256
D = 1024
F = 4096
MOE_D = 2048


def _gmm_body(meta_ref, xd_ref, w13_hbm, w2_hbm, wrow_ref, y_ref,
              w13_buf, w2_buf, sem13, sem2):
    i = pl.program_id(0)
    e = meta_ref[0, i]
    p = meta_ref[1, i]
    first = meta_ref[2, i]
    nxt = meta_ref[3, i]
    hasn = meta_ref[4, i]

    @pl.when(i == 0)
    def _():
        pltpu.make_async_copy(w13_hbm.at[e], w13_buf.at[p], sem13.at[p]).start()
        pltpu.make_async_copy(w2_hbm.at[e], w2_buf.at[p], sem2.at[p]).start()

    @pl.when((first == 1) & (hasn == 1))
    def _():
        q = 1 - p
        pltpu.make_async_copy(w13_hbm.at[nxt], w13_buf.at[q], sem13.at[q]).start()
        pltpu.make_async_copy(w2_hbm.at[nxt], w2_buf.at[q], sem2.at[q]).start()

    @pl.when(first == 1)
    def _():
        pltpu.make_async_copy(w13_hbm.at[e], w13_buf.at[p], sem13.at[p]).wait()
        pltpu.make_async_copy(w2_hbm.at[e], w2_buf.at[p], sem2.at[p]).wait()

    h = jnp.dot(xd_ref[...], w13_buf[p], preferred_element_type=jnp.float32)
    gate = h[:, :MOE_D]
    up = h[:, MOE_D:]
    a = jax.nn.silu(gate) * up
    y = jnp.dot(a, w2_buf[p], preferred_element_type=jnp.float32)
    y_ref[...] = y * wrow_ref[...]


def _seg_meta(eot, nt):
    idx = jnp.arange(nt, dtype=jnp.int32)
    prev = jnp.concatenate([eot[:1] - 1, eot[:-1]])
    first = (eot != prev).astype(jnp.int32)
    seg = jnp.cumsum(first) - 1
    par = (seg % 2).astype(jnp.int32)
    diff = (eot[None, :] != eot[:, None]) & (idx[None, :] > idx[:, None])
    hasn = jnp.any(diff, axis=1)
    j = jnp.argmax(diff, axis=1)
    nxt = jnp.where(hasn, eot[j], eot)
    return jnp.stack([eot, par, first, nxt, hasn.astype(jnp.int32)])


def _grouped_mlp(eot, xd, w13, w2, wrow, nt):
    meta = _seg_meta(eot, nt)
    grid_spec = pltpu.PrefetchScalarGridSpec(
        num_scalar_prefetch=1,
        grid=(nt,),
        in_specs=[
            pl.BlockSpec((TILE, D), lambda i, meta: (i, 0)),
            pl.BlockSpec(memory_space=pl.ANY),
            pl.BlockSpec(memory_space=pl.ANY),
            pl.BlockSpec((TILE, 1), lambda i, meta: (i, 0)),
        ],
        out_specs=pl.BlockSpec((TILE, D), lambda i, meta: (i, 0)),
        scratch_shapes=[
            pltpu.VMEM((2, D, F), jnp.float32),
            pltpu.VMEM((2, MOE_D, D), jnp.float32),
            pltpu.SemaphoreType.DMA((2,)),
            pltpu.SemaphoreType.DMA((2,)),
        ],
    )
    return pl.pallas_call(
        _gmm_body,
        grid_spec=grid_spec,
        out_shape=jax.ShapeDtypeStruct((nt * TILE, D), jnp.float32),
        compiler_params=pltpu.CompilerParams(
            vmem_limit_bytes=110 * 1024 * 1024,
        ),
    )(meta, xd, w13, w2, wrow)


def _routing_body(x_ref, rp_ref, e1_ref, e2_ref, w1_ref, w2_ref):
    l = jnp.dot(x_ref[...], rp_ref[...], preferred_element_type=jnp.float32)
    col = jax.lax.broadcasted_iota(jnp.int32, l.shape, 1)
    lm = jnp.where(col < E, l, -jnp.inf)
    m1 = jnp.max(lm, axis=1, keepdims=True)
    i1 = jnp.min(jnp.where(lm == m1, col, 128), axis=1, keepdims=True)
    l2 = jnp.where(col == i1, -jnp.inf, lm)
    m2 = jnp.max(l2, axis=1, keepdims=True)
    i2 = jnp.min(jnp.where(l2 == m2, col, 128), axis=1, keepdims=True)
    s = jnp.exp(m2 - m1)
    w1 = 1.0 / (1.0 + s)
    e1_ref[...] = i1
    e2_ref[...] = i2
    w1_ref[...] = w1
    w2_ref[...] = 1.0 - w1


def _routing(x_flat, router, tokens):
    rp = jnp.zeros((D, 128), jnp.float32).at[:, :E].set(router)
    return pl.pallas_call(
        _routing_body,
        out_shape=[
            jax.ShapeDtypeStruct((tokens, 1), jnp.int32),
            jax.ShapeDtypeStruct((tokens, 1), jnp.int32),
            jax.ShapeDtypeStruct((tokens, 1), jnp.float32),
            jax.ShapeDtypeStruct((tokens, 1), jnp.float32),
        ],
    )(x_flat, rp)


_NC = 2
_NS = 16
_NW = _NC * _NS


def _sc_collect(y, pos, tokens):
    per_w = tokens // _NW
    mesh = plsc.VectorSubcoreMesh(core_axis_name="c", subcore_axis_name="s")

    @functools.partial(
        pl.kernel, mesh=mesh,
        out_type=jax.ShapeDtypeStruct((tokens, D), jnp.float32),
        scratch_types=[
            pltpu.VMEM((32,), jnp.int32),
            pltpu.VMEM((32,), jnp.int32),
            pltpu.VMEM((32, D), jnp.float32),
            pltpu.VMEM((32, D), jnp.float32),
            pltpu.VMEM((32, D), jnp.float32),
            pltpu.SemaphoreType.DMA,
        ],
    )
    def k(y_hbm, pos_hbm, out_hbm, p1_v, p2_v, r1_v, r2_v, o_v, sem):
        wid = lax.axis_index("s") * _NC + lax.axis_index("c")
        base = wid * per_w
        for c in range(per_w // 32):
            tok0 = base + c * 32
            pltpu.sync_copy(pos_hbm.at[pl.ds(tok0, 32)], p1_v)
            pltpu.sync_copy(pos_hbm.at[pl.ds(tokens + tok0, 32)], p2_v)
            g1 = pltpu.make_async_copy(y_hbm.at[p1_v], r1_v, sem)
            g2 = pltpu.make_async_copy(y_hbm.at[p2_v], r2_v, sem)
            g1.start()
            g2.start()
            g1.wait()
            g2.wait()

            def rbody(r, carry):
                for l in range(D // 16):
                    sl = pl.ds(l * 16, 16)
                    o_v[r, sl] = r1_v[r, sl] + r2_v[r, sl]
                return carry

            lax.fori_loop(0, 32, rbody, 0)
            pltpu.sync_copy(o_v, out_hbm.at[pl.ds(tok0, 32)])

    return k(y, pos)


def kernel(x, moe_router, moe_w13, moe_w2):
    b, s, d = x.shape
    tokens = b * s
    x_flat = x.reshape(tokens, d)
    nt = tokens * TOPK // TILE + E - 1
    r_pad = nt * TILE

    e1, e2, w1, w2 = _routing(x_flat, moe_router, tokens)

    e_slot = jnp.concatenate([e1[:, 0], e2[:, 0]])
    w_slot = jnp.concatenate([w1[:, 0], w2[:, 0]])
    oh = (e_slot[:, None] == jnp.arange(E)[None, :]).astype(jnp.int32)
    pref = jnp.cumsum(oh, axis=0)
    cnt = pref[-1]
    padded = ((cnt + TILE - 1) // TILE) * TILE
    pad_off = jnp.concatenate([jnp.zeros((1,), jnp.int32),
                               jnp.cumsum(padded)[:-1].astype(jnp.int32)])
    rank = jnp.sum(pref * oh, axis=1) - 1
    pos = (pad_off[e_slot] + rank).astype(jnp.int32)
    src = jnp.zeros((r_pad,), jnp.int32).at[pos].set(
        jnp.arange(tokens * TOPK, dtype=jnp.int32) % tokens)
    wdisp = jnp.zeros((r_pad,), jnp.float32).at[pos].set(w_slot)
    pad_end = (pad_off + padded).astype(jnp.int32)
    tile_start = jnp.arange(nt, dtype=jnp.int32) * TILE
    eot = jnp.minimum(
        jnp.sum((pad_end[None, :] <= tile_start[:, None]).astype(jnp.int32),
                axis=1), E - 1).astype(jnp.int32)

    xd = x_flat[src]

    y = _grouped_mlp(eot, xd, moe_w13, moe_w2, wdisp[:, None], nt)

    out_flat = _sc_collect(y, pos, tokens)
    return out_flat.reshape(b, s, d)
